# Initial kernel scaffold; baseline (speedup 1.0000x reference)
import functools

import jax
import jax.numpy as jnp
from jax import lax
from jax.experimental import pallas as pl
from jax.experimental.pallas import tpu as pltpu
from jax.experimental.pallas import tpu_sc as plsc

N = 10000
D = 128
DP = 144
E = 320000
NC, NS = 2, 16
NW = NC * NS
EW = E // NW
CHUNK = 80
CHUNKS = EW // CHUNK
RPS = N // NS
BM = 1000


def _pad_cols(m):
    col = lax.broadcasted_iota(jnp.int32, (m, DP - D), 1)
    return jnp.where(col == 0, 1.0, 0.0).astype(jnp.float32)


def _mm_pad_body(x_ref, w_ref, b_ref, o_ref):
    y = jnp.dot(x_ref[...], w_ref[...], preferred_element_type=jnp.float32)
    o_ref[:, :D] = y + b_ref[...]
    o_ref[:, D:] = _pad_cols(o_ref.shape[0])


def _mm_pad(x, w, b):
    return pl.pallas_call(
        _mm_pad_body,
        grid=(N // BM,),
        in_specs=[
            pl.BlockSpec((BM, D), lambda i: (i, 0)),
            pl.BlockSpec((D, D), lambda i: (0, 0)),
            pl.BlockSpec((1, D), lambda i: (0, 0)),
        ],
        out_specs=pl.BlockSpec((BM, DP), lambda i: (i, 0)),
        out_shape=jax.ShapeDtypeStruct((N, DP), jnp.float32),
    )(x, w, b.reshape(1, D))


def _combine_mm_pad_body(a0_ref, a1_ref, w_ref, b_ref, o_ref):
    t = a0_ref[...] + a1_ref[...]
    cnt = jnp.maximum(t[:, D:D + 1], 1.0)
    h = t[:, :D] / cnt
    h = jnp.where(h >= 0.0, h, 0.01 * h)
    y = jnp.dot(h, w_ref[...], preferred_element_type=jnp.float32)
    o_ref[:, :D] = y + b_ref[...]
    o_ref[:, D:] = _pad_cols(o_ref.shape[0])


def _combine_mm_pad(a0, a1, w, b):
    return pl.pallas_call(
        _combine_mm_pad_body,
        grid=(N // BM,),
        in_specs=[
            pl.BlockSpec((BM, DP), lambda i: (i, 0)),
            pl.BlockSpec((BM, DP), lambda i: (i, 0)),
            pl.BlockSpec((D, D), lambda i: (0, 0)),
            pl.BlockSpec((1, D), lambda i: (0, 0)),
        ],
        out_specs=pl.BlockSpec((BM, DP), lambda i: (i, 0)),
        out_shape=jax.ShapeDtypeStruct((N, DP), jnp.float32),
    )(a0, a1, w, b.reshape(1, D))


def _finalize_body(a0_ref, a1_ref, o_ref):
    t = a0_ref[...] + a1_ref[...]
    o_ref[...] = t[:, :D] / jnp.maximum(t[:, D:D + 1], 1.0)


def _finalize(a0, a1):
    return pl.pallas_call(
        _finalize_body,
        grid=(N // BM,),
        in_specs=[
            pl.BlockSpec((BM, DP), lambda i: (i, 0)),
            pl.BlockSpec((BM, DP), lambda i: (i, 0)),
        ],
        out_specs=pl.BlockSpec((BM, D), lambda i: (i, 0)),
        out_shape=jax.ShapeDtypeStruct((N, D), jnp.float32),
    )(a0, a1)


_MESH = plsc.VectorSubcoreMesh(
    core_axis_name="c", subcore_axis_name="s", num_cores=NC, num_subcores=NS)


@functools.partial(
    pl.kernel,
    out_type=jax.ShapeDtypeStruct((NC, N, DP), jnp.float32),
    mesh=_MESH,
    scratch_types=[
        pltpu.VMEM((CHUNKS, CHUNK), jnp.int32),
        pltpu.VMEM((CHUNKS, CHUNK), jnp.int32),
        pltpu.VMEM((CHUNK, DP), jnp.float32),
        pltpu.VMEM_SHARED((N, DP), jnp.float32),
        pltpu.SemaphoreType.DMA,
        pltpu.SemaphoreType.DMA,
    ],
)
def _sc_segsum(table_hbm, src_hbm, dst_hbm, zeros_hbm, out_hbm,
               src_v, dst_v, rows_v, acc_sh, gsem, ssem):
    c = lax.axis_index("c")
    s = lax.axis_index("s")
    w = s * NC + c

    pltpu.sync_copy(zeros_hbm, acc_sh.at[pl.ds(s * RPS, RPS)])
    pltpu.sync_copy(src_hbm.at[pl.ds(w * CHUNKS, CHUNKS)], src_v)
    pltpu.sync_copy(dst_hbm.at[pl.ds(w * CHUNKS, CHUNKS)], dst_v)
    plsc.subcore_barrier()

    def body(j, carry):
        pltpu.async_copy(table_hbm.at[src_v.at[j]], rows_v, gsem).wait()
        pltpu.async_copy(rows_v, acc_sh.at[dst_v.at[j]], ssem, add=True).wait()
        return carry

    lax.fori_loop(0, CHUNKS, body, 0)
    plsc.subcore_barrier()
    pltpu.sync_copy(acc_sh.at[pl.ds(s * RPS, RPS)],
                    out_hbm.at[c, pl.ds(s * RPS, RPS)])


def kernel(embed_chemical, embed_gene, W1_affects, b1_affects, W1_binds,
           b1_binds, W2_affects, b2_affects, W2_binds, b2_binds,
           edge_affects, edge_binds):
    src1 = edge_binds[0].astype(jnp.int32).reshape(NW * CHUNKS, CHUNK)
    dst1 = edge_binds[1].astype(jnp.int32).reshape(NW * CHUNKS, CHUNK)
    src2 = edge_affects[0].astype(jnp.int32).reshape(NW * CHUNKS, CHUNK)
    dst2 = edge_affects[1].astype(jnp.int32).reshape(NW * CHUNKS, CHUNK)
    zeros = jnp.zeros((RPS, DP), jnp.float32)

    wh1 = _mm_pad(embed_chemical, W1_binds, b1_binds)
    acc1 = _sc_segsum(wh1, src1, dst1, zeros)
    wh2 = _combine_mm_pad(acc1[0], acc1[1], W2_affects, b2_affects)
    acc2 = _sc_segsum(wh2, src2, dst2, zeros)
    return _finalize(acc2[0], acc2[1])


# speedup vs baseline: 12.4656x; 12.4656x over previous
import functools

import jax
import jax.numpy as jnp
from jax import lax
from jax.experimental import pallas as pl
from jax.experimental.pallas import tpu as pltpu
from jax.experimental.pallas import tpu_sc as plsc

N = 10000
D = 128
E = 320000
NC, NS = 2, 16
NW = NC * NS
CHUNK = 128
EROWS = E // CHUNK
RPW = EROWS // NW
XTRA = EROWS - NW * RPW
RL = 80
NP = 10112
NPB = NP // D
RPS = NP // NS
TRASH = NP - 1
BM = 1000
L = 16
PH = 24
PHASES = (PH, PH, PH, RL - 3 * PH)


def _mm_body(x_ref, w_ref, b_ref, o_ref):
    y = jnp.dot(x_ref[...], w_ref[...], preferred_element_type=jnp.float32)
    o_ref[...] = y + b_ref[...]


def _mm(x, w, b):
    return pl.pallas_call(
        _mm_body,
        grid=(N // BM,),
        in_specs=[
            pl.BlockSpec((BM, D), lambda i: (i, 0)),
            pl.BlockSpec((D, D), lambda i: (0, 0)),
            pl.BlockSpec((1, D), lambda i: (0, 0)),
        ],
        out_specs=pl.BlockSpec((BM, D), lambda i: (i, 0)),
        out_shape=jax.ShapeDtypeStruct((N, D), jnp.float32),
    )(x, w, b.reshape(1, D))


def _inv_cnt_col(cnt3):
    csum = jnp.maximum(jnp.sum(cnt3, axis=0), 1.0)
    inv = (1.0 / csum).reshape(NPB, 1, D)
    invT = lax.dot_general(
        inv, jnp.ones((NPB, 1, 1), jnp.float32),
        dimension_numbers=(((1,), (1,)), ((0,), (0,))),
        preferred_element_type=jnp.float32)
    return invT.reshape(NP, 1)


def _combine_mm_body(acc_ref, cnt_ref, w_ref, b_ref, o_ref):
    t = acc_ref[0] + acc_ref[1]
    h = t * _inv_cnt_col(cnt_ref[...])
    h = jnp.where(h >= 0.0, h, 0.01 * h)
    y = jnp.dot(h, w_ref[...], preferred_element_type=jnp.float32)
    o_ref[...] = y + b_ref[...]


def _combine_mm(acc, cnt3, w, b):
    return pl.pallas_call(
        _combine_mm_body,
        out_shape=jax.ShapeDtypeStruct((NP, D), jnp.float32),
    )(acc, cnt3, w, b.reshape(1, D))


def _finalize_body(acc_ref, cnt_ref, o_ref):
    t = acc_ref[0] + acc_ref[1]
    o_ref[...] = t * _inv_cnt_col(cnt_ref[...])


def _finalize(acc, cnt3):
    return pl.pallas_call(
        _finalize_body,
        out_shape=jax.ShapeDtypeStruct((NP, D), jnp.float32),
    )(acc, cnt3)


_MESH = plsc.VectorSubcoreMesh(
    core_axis_name="c", subcore_axis_name="s", num_cores=NC, num_subcores=NS)


@functools.partial(
    pl.kernel,
    out_type=[
        jax.ShapeDtypeStruct((NC, NP, D), jnp.float32),
        jax.ShapeDtypeStruct((NW, NPB, D), jnp.float32),
    ],
    mesh=_MESH,
    compiler_params=pltpu.CompilerParams(needs_layout_passes=False),
    scratch_types=[
        pltpu.VMEM((RL,), jnp.int32),
        pltpu.VMEM((PH, CHUNK), jnp.int32),
        pltpu.VMEM((PH, CHUNK), jnp.int32),
        pltpu.VMEM((CHUNK, D), jnp.float32),
        pltpu.VMEM((CHUNK, D), jnp.float32),
        pltpu.VMEM((NPB, D), jnp.float32),
        pltpu.VMEM_SHARED((NP, D), jnp.float32),
        pltpu.SemaphoreType.DMA,
        pltpu.SemaphoreType.DMA,
        pltpu.SemaphoreType.DMA,
        pltpu.SemaphoreType.DMA,
    ],
)
def _sc_segsum(table_hbm, src_hbm, dst_hbm, acc_out, cnt_out,
               rlist_v, src_v, dst_v, rows_a, rows_b, cnt_v, acc_sh,
               gsa, gsb, ssa, ssb):
    gsem = gsa
    rows_v = rows_a
    c = lax.axis_index("c")
    s = lax.axis_index("s")
    w = s * NC + c
    nrows = RPW + jnp.where(w < XTRA, 1, 0)

    iota = lax.iota(jnp.int32, L)
    for k in range(RL // L):
        slot = k * L + iota
        vals = jnp.where(
            slot < RPW, RPW * w + slot,
            jnp.where((slot == RPW) & (w < XTRA), NW * RPW + w, 0))
        rlist_v[pl.ds(k * L, L)] = vals

    def zrows_body(i, carry):
        rows_v[i // (D // L), pl.ds((i % (D // L)) * L, L)] = (
            jnp.zeros((L,), jnp.float32))
        return carry

    lax.fori_loop(0, CHUNK * (D // L), zrows_body, 0)

    def zcnt_body(i, carry):
        cnt_v[i // (D // L), pl.ds((i % (D // L)) * L, L)] = (
            jnp.zeros((L,), jnp.float32))
        return carry

    lax.fori_loop(0, NPB * (D // L), zcnt_body, 0)

    def zacc_body(z, carry):
        pltpu.sync_copy(rows_v.at[pl.ds(0, CHUNK)],
                        acc_sh.at[pl.ds(s * RPS + z * CHUNK, CHUNK)])
        return carry

    lax.fori_loop(0, RPS // CHUNK, zacc_body, 0)
    rem = RPS % CHUNK
    pltpu.sync_copy(rows_v.at[pl.ds(0, rem)],
                    acc_sh.at[pl.ds(s * RPS + (RPS // CHUNK) * CHUNK, rem)])
    plsc.subcore_barrier()

    def _hist(j):
        for k in range(CHUNK // L):
            vec = dst_v[j, pl.ds(k * L, L)]
            cnts, last = plsc.scan_count(vec)
            plsc.addupdate_scatter(
                cnt_v, [vec >> 7, vec & 127], cnts.astype(jnp.float32),
                mask=last)

    base = 0
    for pi, ph in enumerate(PHASES):
        nloc = ph if pi < len(PHASES) - 1 else nrows - base
        pltpu.async_copy(
            src_hbm.at[rlist_v.at[pl.ds(base, ph)]],
            src_v.at[pl.ds(0, ph)], gsem)
        pltpu.async_copy(
            dst_hbm.at[rlist_v.at[pl.ds(base, ph)]],
            dst_v.at[pl.ds(0, ph)], gsem)
        pltpu.make_async_copy(
            src_hbm.at[rlist_v.at[pl.ds(base, ph)]],
            src_v.at[pl.ds(0, ph)], gsem).wait()
        pltpu.make_async_copy(
            dst_hbm.at[rlist_v.at[pl.ds(base, ph)]],
            dst_v.at[pl.ds(0, ph)], gsem).wait()

        pltpu.async_copy(table_hbm.at[src_v.at[0]], rows_a, gsa)

        def body(p, carry):
            j0 = 2 * p
            j1 = j0 + 1

            @pl.when(j1 < nloc)
            def _():
                pltpu.async_copy(table_hbm.at[src_v.at[j1]], rows_b, gsb)

            pltpu.make_async_copy(
                table_hbm.at[src_v.at[0]], rows_a, gsa).wait()
            _hist(j0)
            pltpu.async_copy(rows_a, acc_sh.at[dst_v.at[j0]], ssa, add=True)
            pltpu.make_async_copy(rows_a, acc_sh.at[dst_v.at[0]], ssa).wait()

            @pl.when(j0 + 2 < nloc)
            def _():
                pltpu.async_copy(
                    table_hbm.at[src_v.at[j0 + 2]], rows_a, gsa)

            @pl.when(j1 < nloc)
            def _():
                pltpu.make_async_copy(
                    table_hbm.at[src_v.at[0]], rows_b, gsb).wait()
                _hist(j1)
                pltpu.async_copy(
                    rows_b, acc_sh.at[dst_v.at[j1]], ssb, add=True)
                pltpu.make_async_copy(
                    rows_b, acc_sh.at[dst_v.at[0]], ssb).wait()

            return carry

        lax.fori_loop(0, (nloc + 1) // 2, body, 0)
        base += ph
    plsc.subcore_barrier()
    pltpu.sync_copy(acc_sh.at[pl.ds(s * RPS, RPS)],
                    acc_out.at[c, pl.ds(s * RPS, RPS)])
    pltpu.sync_copy(cnt_v, cnt_out.at[w])


def kernel(embed_chemical, embed_gene, W1_affects, b1_affects, W1_binds,
           b1_binds, W2_affects, b2_affects, W2_binds, b2_binds,
           edge_affects, edge_binds):
    src1 = edge_binds[0].astype(jnp.int32).reshape(EROWS, CHUNK)
    dst1 = edge_binds[1].astype(jnp.int32).reshape(EROWS, CHUNK)
    src2 = edge_affects[0].astype(jnp.int32).reshape(EROWS, CHUNK)
    dst2 = edge_affects[1].astype(jnp.int32).reshape(EROWS, CHUNK)

    wh1 = _mm(embed_chemical, W1_binds, b1_binds)
    acc1, cnt1 = _sc_segsum(wh1, src1, dst1)
    wh2 = _combine_mm(acc1, cnt1, W2_affects, b2_affects)
    acc2, cnt2 = _sc_segsum(wh2, src2, dst2)
    out = _finalize(acc2, cnt2)
    return out[:N]


# speedup vs baseline: 12.6033x; 1.0111x over previous
import functools

import jax
import jax.numpy as jnp
from jax import lax
from jax.experimental import pallas as pl
from jax.experimental.pallas import tpu as pltpu
from jax.experimental.pallas import tpu_sc as plsc

N = 10000
D = 128
E = 320000
NC, NS = 2, 16
NW = NC * NS
CHUNK = 128
EROWS = E // CHUNK
RPW = EROWS // NW
XTRA = EROWS - NW * RPW
RL = 80
NP = 10112
NPB = NP // D
RPS = NP // NS
TRASH = NP - 1
BM = 1000
L = 16
PH = 24
PHASES = (PH, PH, PH, RL - 3 * PH)


def _mm_body(x_ref, w_ref, b_ref, o_ref):
    y = jnp.dot(x_ref[...], w_ref[...], preferred_element_type=jnp.float32)
    o_ref[...] = y + b_ref[...]


def _mm(x, w, b):
    return pl.pallas_call(
        _mm_body,
        grid=(N // BM,),
        in_specs=[
            pl.BlockSpec((BM, D), lambda i: (i, 0)),
            pl.BlockSpec((D, D), lambda i: (0, 0)),
            pl.BlockSpec((1, D), lambda i: (0, 0)),
        ],
        out_specs=pl.BlockSpec((BM, D), lambda i: (i, 0)),
        out_shape=jax.ShapeDtypeStruct((N, D), jnp.float32),
    )(x, w, b.reshape(1, D))


def _inv_cnt_col(cnt3):
    csum = jnp.maximum(jnp.sum(cnt3, axis=0), 1.0)
    inv = (1.0 / csum).reshape(NPB, 1, D)
    invT = lax.dot_general(
        inv, jnp.ones((NPB, 1, 1), jnp.float32),
        dimension_numbers=(((1,), (1,)), ((0,), (0,))),
        preferred_element_type=jnp.float32)
    return invT.reshape(NP, 1)


def _combine_mm_body(acc_ref, cnt_ref, w_ref, b_ref, o_ref):
    t = acc_ref[0] + acc_ref[1]
    h = t * _inv_cnt_col(cnt_ref[...])
    h = jnp.where(h >= 0.0, h, 0.01 * h)
    y = jnp.dot(h, w_ref[...], preferred_element_type=jnp.float32)
    o_ref[...] = y + b_ref[...]


def _combine_mm(acc, cnt3, w, b):
    return pl.pallas_call(
        _combine_mm_body,
        out_shape=jax.ShapeDtypeStruct((NP, D), jnp.float32),
    )(acc, cnt3, w, b.reshape(1, D))


def _finalize_body(acc_ref, cnt_ref, o_ref):
    t = acc_ref[0] + acc_ref[1]
    o_ref[...] = t * _inv_cnt_col(cnt_ref[...])


def _finalize(acc, cnt3):
    return pl.pallas_call(
        _finalize_body,
        out_shape=jax.ShapeDtypeStruct((NP, D), jnp.float32),
    )(acc, cnt3)


_MESH = plsc.VectorSubcoreMesh(
    core_axis_name="c", subcore_axis_name="s", num_cores=NC, num_subcores=NS)


@functools.partial(
    pl.kernel,
    out_type=[
        jax.ShapeDtypeStruct((NC, NP, D), jnp.float32),
        jax.ShapeDtypeStruct((NW, NPB, D), jnp.float32),
    ],
    mesh=_MESH,
    compiler_params=pltpu.CompilerParams(needs_layout_passes=False),
    scratch_types=[
        pltpu.VMEM((RL,), jnp.int32),
        pltpu.VMEM((PH, CHUNK), jnp.int32),
        pltpu.VMEM((PH, CHUNK), jnp.int32),
        pltpu.VMEM((CHUNK, D), jnp.float32),
        pltpu.VMEM((CHUNK, D), jnp.float32),
        pltpu.VMEM((NPB, D), jnp.float32),
        pltpu.VMEM_SHARED((NP, D), jnp.float32),
        pltpu.SemaphoreType.DMA,
        pltpu.SemaphoreType.DMA,
        pltpu.SemaphoreType.DMA,
        pltpu.SemaphoreType.DMA,
    ],
)
def _sc_segsum(table_hbm, src_hbm, dst_hbm, acc_out, cnt_out,
               rlist_v, src_v, dst_v, rows_a, rows_b, cnt_v, acc_sh,
               gsa, gsb, ssa, ssb):
    gsem = gsa
    rows_v = rows_a
    c = lax.axis_index("c")
    s = lax.axis_index("s")
    w = s * NC + c
    nrows = RPW + jnp.where(w < XTRA, 1, 0)

    iota = lax.iota(jnp.int32, L)
    for k in range(RL // L):
        slot = k * L + iota
        vals = jnp.where(
            slot < RPW, RPW * w + slot,
            jnp.where((slot == RPW) & (w < XTRA), NW * RPW + w, 0))
        rlist_v[pl.ds(k * L, L)] = vals

    def zrows_body(i, carry):
        rows_v[i // (D // L), pl.ds((i % (D // L)) * L, L)] = (
            jnp.zeros((L,), jnp.float32))
        return carry

    lax.fori_loop(0, CHUNK * (D // L), zrows_body, 0)

    def zcnt_body(i, carry):
        cnt_v[i // (D // L), pl.ds((i % (D // L)) * L, L)] = (
            jnp.zeros((L,), jnp.float32))
        return carry

    lax.fori_loop(0, NPB * (D // L), zcnt_body, 0)

    def zacc_body(z, carry):
        pltpu.sync_copy(rows_v.at[pl.ds(0, CHUNK)],
                        acc_sh.at[pl.ds(s * RPS + z * CHUNK, CHUNK)])
        return carry

    lax.fori_loop(0, RPS // CHUNK, zacc_body, 0)
    rem = RPS % CHUNK
    pltpu.sync_copy(rows_v.at[pl.ds(0, rem)],
                    acc_sh.at[pl.ds(s * RPS + (RPS // CHUNK) * CHUNK, rem)])
    plsc.subcore_barrier()

    def _hist(j):
        for k in range(CHUNK // L):
            vec = dst_v[j, pl.ds(k * L, L)]
            cnts, last = plsc.scan_count(vec)
            plsc.addupdate_scatter(
                cnt_v, [vec >> 7, vec & 127], cnts.astype(jnp.float32),
                mask=last)

    base = 0
    for pi, ph in enumerate(PHASES):
        nloc = ph if pi < len(PHASES) - 1 else nrows - base
        pltpu.async_copy(
            src_hbm.at[rlist_v.at[pl.ds(base, ph)]],
            src_v.at[pl.ds(0, ph)], gsem)
        pltpu.async_copy(
            dst_hbm.at[rlist_v.at[pl.ds(base, ph)]],
            dst_v.at[pl.ds(0, ph)], gsem)
        pltpu.make_async_copy(
            src_hbm.at[rlist_v.at[pl.ds(base, ph)]],
            src_v.at[pl.ds(0, ph)], gsem).wait()
        pltpu.make_async_copy(
            dst_hbm.at[rlist_v.at[pl.ds(base, ph)]],
            dst_v.at[pl.ds(0, ph)], gsem).wait()

        pltpu.async_copy(table_hbm.at[src_v.at[0]], rows_a, gsa)

        def body(p, carry):
            j0 = 2 * p
            j1 = j0 + 1

            @pl.when(j1 < nloc)
            def _():
                pltpu.async_copy(table_hbm.at[src_v.at[j1]], rows_b, gsb)

            pltpu.make_async_copy(
                table_hbm.at[src_v.at[0]], rows_a, gsa).wait()
            pltpu.async_copy(rows_a, acc_sh.at[dst_v.at[j0]], ssa, add=True)
            _hist(j0)
            pltpu.make_async_copy(rows_a, acc_sh.at[dst_v.at[0]], ssa).wait()

            @pl.when(j0 + 2 < nloc)
            def _():
                pltpu.async_copy(
                    table_hbm.at[src_v.at[j0 + 2]], rows_a, gsa)

            @pl.when(j1 < nloc)
            def _():
                pltpu.make_async_copy(
                    table_hbm.at[src_v.at[0]], rows_b, gsb).wait()
                pltpu.async_copy(
                    rows_b, acc_sh.at[dst_v.at[j1]], ssb, add=True)
                _hist(j1)
                pltpu.make_async_copy(
                    rows_b, acc_sh.at[dst_v.at[0]], ssb).wait()

            return carry

        lax.fori_loop(0, (nloc + 1) // 2, body, 0)
        base += ph
    plsc.subcore_barrier()
    pltpu.sync_copy(acc_sh.at[pl.ds(s * RPS, RPS)],
                    acc_out.at[c, pl.ds(s * RPS, RPS)])
    pltpu.sync_copy(cnt_v, cnt_out.at[w])


def kernel(embed_chemical, embed_gene, W1_affects, b1_affects, W1_binds,
           b1_binds, W2_affects, b2_affects, W2_binds, b2_binds,
           edge_affects, edge_binds):
    src1 = edge_binds[0].astype(jnp.int32).reshape(EROWS, CHUNK)
    dst1 = edge_binds[1].astype(jnp.int32).reshape(EROWS, CHUNK)
    src2 = edge_affects[0].astype(jnp.int32).reshape(EROWS, CHUNK)
    dst2 = edge_affects[1].astype(jnp.int32).reshape(EROWS, CHUNK)

    wh1 = _mm(embed_chemical, W1_binds, b1_binds)
    acc1, cnt1 = _sc_segsum(wh1, src1, dst1)
    wh2 = _combine_mm(acc1, cnt1, W2_affects, b2_affects)
    acc2, cnt2 = _sc_segsum(wh2, src2, dst2)
    out = _finalize(acc2, cnt2)
    return out[:N]


# speedup vs baseline: 12.6503x; 1.0037x over previous
import functools

import jax
import jax.numpy as jnp
from jax import lax
from jax.experimental import pallas as pl
from jax.experimental.pallas import tpu as pltpu
from jax.experimental.pallas import tpu_sc as plsc

N = 10000
D = 128
E = 320000
NC, NS = 2, 16
NW = NC * NS
CHUNK = 128
EROWS = E // CHUNK
RPW = EROWS // NW
XTRA = EROWS - NW * RPW
RL = 80
NP = 10112
NPB = NP // D
RPS = NP // NS
TRASH = NP - 1
BM = 1000
L = 16
PH = 24
PHASES = (PH, PH, PH, RL - 3 * PH)


def _mm_body(x_ref, w_ref, b_ref, o_ref):
    y = jnp.dot(x_ref[...], w_ref[...], preferred_element_type=jnp.float32)
    o_ref[...] = y + b_ref[...]


def _mm(x, w, b):
    return pl.pallas_call(
        _mm_body,
        grid=(N // BM,),
        in_specs=[
            pl.BlockSpec((BM, D), lambda i: (i, 0)),
            pl.BlockSpec((D, D), lambda i: (0, 0)),
            pl.BlockSpec((1, D), lambda i: (0, 0)),
        ],
        out_specs=pl.BlockSpec((BM, D), lambda i: (i, 0)),
        out_shape=jax.ShapeDtypeStruct((N, D), jnp.float32),
    )(x, w, b.reshape(1, D))


def _inv_cnt_col(cnt3):
    csum = jnp.maximum(jnp.sum(cnt3, axis=0), 1.0)
    inv = (1.0 / csum).reshape(NPB, 1, D)
    invT = lax.dot_general(
        inv, jnp.ones((NPB, 1, 1), jnp.float32),
        dimension_numbers=(((1,), (1,)), ((0,), (0,))),
        preferred_element_type=jnp.float32)
    return invT.reshape(NP, 1)


def _combine_mm_body(acc_ref, cnt_ref, w_ref, b_ref, o_ref):
    t = acc_ref[0] + acc_ref[1]
    h = t * _inv_cnt_col(cnt_ref[...])
    h = jnp.where(h >= 0.0, h, 0.01 * h)
    y = jnp.dot(h, w_ref[...], preferred_element_type=jnp.float32)
    o_ref[...] = y + b_ref[...]


def _combine_mm(acc, cnt3, w, b):
    return pl.pallas_call(
        _combine_mm_body,
        out_shape=jax.ShapeDtypeStruct((NP, D), jnp.float32),
    )(acc, cnt3, w, b.reshape(1, D))


def _finalize_body(acc_ref, cnt_ref, o_ref):
    t = acc_ref[0] + acc_ref[1]
    o_ref[...] = t * _inv_cnt_col(cnt_ref[...])


def _finalize(acc, cnt3):
    return pl.pallas_call(
        _finalize_body,
        out_shape=jax.ShapeDtypeStruct((NP, D), jnp.float32),
    )(acc, cnt3)


_MESH = plsc.VectorSubcoreMesh(
    core_axis_name="c", subcore_axis_name="s", num_cores=NC, num_subcores=NS)


@functools.partial(
    pl.kernel,
    out_type=[
        jax.ShapeDtypeStruct((NC, NP, D), jnp.float32),
        jax.ShapeDtypeStruct((NW, NPB, D), jnp.float32),
    ],
    mesh=_MESH,
    compiler_params=pltpu.CompilerParams(needs_layout_passes=False),
    scratch_types=[
        pltpu.VMEM((RL,), jnp.int32),
        pltpu.VMEM((PH, CHUNK), jnp.int32),
        pltpu.VMEM((PH, CHUNK), jnp.int32),
        pltpu.VMEM((CHUNK, D), jnp.float32),
        pltpu.VMEM((CHUNK, D), jnp.float32),
        pltpu.VMEM((NPB, D), jnp.float32),
        pltpu.VMEM_SHARED((NP, D), jnp.float32),
        pltpu.SemaphoreType.DMA,
        pltpu.SemaphoreType.DMA,
        pltpu.SemaphoreType.DMA,
        pltpu.SemaphoreType.DMA,
    ],
)
def _sc_segsum(table_hbm, src_hbm, dst_hbm, acc_out, cnt_out,
               rlist_v, src_v, dst_v, rows_a, rows_b, cnt_v, acc_sh,
               gsa, gsb, ssa, ssb):
    gsem = gsa
    rows_v = rows_a
    c = lax.axis_index("c")
    s = lax.axis_index("s")
    w = s * NC + c
    nrows = RPW + jnp.where(w < XTRA, 1, 0)

    iota = lax.iota(jnp.int32, L)
    for k in range(RL // L):
        slot = k * L + iota
        vals = jnp.where(
            slot < RPW, RPW * w + slot,
            jnp.where((slot == RPW) & (w < XTRA), NW * RPW + w, 0))
        rlist_v[pl.ds(k * L, L)] = vals

    def zrows_body(i, carry):
        rows_v[i // (D // L), pl.ds((i % (D // L)) * L, L)] = (
            jnp.zeros((L,), jnp.float32))
        return carry

    lax.fori_loop(0, CHUNK * (D // L), zrows_body, 0)

    def zcnt_body(i, carry):
        cnt_v[i // (D // L), pl.ds((i % (D // L)) * L, L)] = (
            jnp.zeros((L,), jnp.float32))
        return carry

    lax.fori_loop(0, NPB * (D // L), zcnt_body, 0)

    rem = RPS % CHUNK
    for z in range(RPS // CHUNK):
        pltpu.async_copy(rows_v.at[pl.ds(0, CHUNK)],
                         acc_sh.at[pl.ds(s * RPS + z * CHUNK, CHUNK)], ssa)
    pltpu.async_copy(rows_v.at[pl.ds(0, rem)],
                     acc_sh.at[pl.ds(s * RPS + (RPS // CHUNK) * CHUNK, rem)],
                     ssb)
    for z in range(RPS // CHUNK):
        pltpu.make_async_copy(
            rows_v.at[pl.ds(0, CHUNK)],
            acc_sh.at[pl.ds(s * RPS + z * CHUNK, CHUNK)], ssa).wait()
    pltpu.make_async_copy(
        rows_v.at[pl.ds(0, rem)],
        acc_sh.at[pl.ds(s * RPS + (RPS // CHUNK) * CHUNK, rem)], ssb).wait()
    plsc.subcore_barrier()

    def _hist(j):
        for k in range(CHUNK // L):
            vec = dst_v[j, pl.ds(k * L, L)]
            cnts, last = plsc.scan_count(vec)
            plsc.addupdate_scatter(
                cnt_v, [vec >> 7, vec & 127], cnts.astype(jnp.float32),
                mask=last)

    base = 0
    for pi, ph in enumerate(PHASES):
        nloc = ph if pi < len(PHASES) - 1 else nrows - base
        pltpu.async_copy(
            src_hbm.at[rlist_v.at[pl.ds(base, ph)]],
            src_v.at[pl.ds(0, ph)], gsem)
        pltpu.async_copy(
            dst_hbm.at[rlist_v.at[pl.ds(base, ph)]],
            dst_v.at[pl.ds(0, ph)], gsem)
        pltpu.make_async_copy(
            src_hbm.at[rlist_v.at[pl.ds(base, ph)]],
            src_v.at[pl.ds(0, ph)], gsem).wait()
        pltpu.make_async_copy(
            dst_hbm.at[rlist_v.at[pl.ds(base, ph)]],
            dst_v.at[pl.ds(0, ph)], gsem).wait()

        pltpu.async_copy(table_hbm.at[src_v.at[0]], rows_a, gsa)

        def body(p, carry):
            j0 = 2 * p
            j1 = j0 + 1

            @pl.when(j1 < nloc)
            def _():
                pltpu.async_copy(table_hbm.at[src_v.at[j1]], rows_b, gsb)

            pltpu.make_async_copy(
                table_hbm.at[src_v.at[0]], rows_a, gsa).wait()
            pltpu.async_copy(rows_a, acc_sh.at[dst_v.at[j0]], ssa, add=True)
            _hist(j0)
            pltpu.make_async_copy(rows_a, acc_sh.at[dst_v.at[0]], ssa).wait()

            @pl.when(j0 + 2 < nloc)
            def _():
                pltpu.async_copy(
                    table_hbm.at[src_v.at[j0 + 2]], rows_a, gsa)

            @pl.when(j1 < nloc)
            def _():
                pltpu.make_async_copy(
                    table_hbm.at[src_v.at[0]], rows_b, gsb).wait()
                pltpu.async_copy(
                    rows_b, acc_sh.at[dst_v.at[j1]], ssb, add=True)
                _hist(j1)
                pltpu.make_async_copy(
                    rows_b, acc_sh.at[dst_v.at[0]], ssb).wait()

            return carry

        lax.fori_loop(0, (nloc + 1) // 2, body, 0)
        base += ph
    plsc.subcore_barrier()
    pltpu.sync_copy(acc_sh.at[pl.ds(s * RPS, RPS)],
                    acc_out.at[c, pl.ds(s * RPS, RPS)])
    pltpu.sync_copy(cnt_v, cnt_out.at[w])


def kernel(embed_chemical, embed_gene, W1_affects, b1_affects, W1_binds,
           b1_binds, W2_affects, b2_affects, W2_binds, b2_binds,
           edge_affects, edge_binds):
    src1 = edge_binds[0].astype(jnp.int32).reshape(EROWS, CHUNK)
    dst1 = edge_binds[1].astype(jnp.int32).reshape(EROWS, CHUNK)
    src2 = edge_affects[0].astype(jnp.int32).reshape(EROWS, CHUNK)
    dst2 = edge_affects[1].astype(jnp.int32).reshape(EROWS, CHUNK)

    wh1 = _mm(embed_chemical, W1_binds, b1_binds)
    acc1, cnt1 = _sc_segsum(wh1, src1, dst1)
    wh2 = _combine_mm(acc1, cnt1, W2_affects, b2_affects)
    acc2, cnt2 = _sc_segsum(wh2, src2, dst2)
    out = _finalize(acc2, cnt2)
    return out[:N]
